# baseline (speedup 1.0000x reference)
import functools

import jax
import jax.numpy as jnp
from jax.experimental import pallas as pl
from jax.experimental.pallas import tpu as pltpu
from jax.experimental.pallas import tpu_sc as plsc

_KNN = 5
_HIGH = jax.lax.Precision.HIGHEST


def _dot(a, b, dims):
    return jax.lax.dot_general(
        a, b, (dims, ((), ())),
        preferred_element_type=jnp.float32, precision=_HIGH)


def _stage1_body(xn_ref, e_ref, w0_ref, g0_ref, b0_ref, anc_ref, sigp_ref,
                 nodes_ref):
    B, N, C = xn_ref.shape

    hs = []
    ssum = jnp.zeros((1, C), jnp.float32)
    qsum = jnp.zeros((1, C), jnp.float32)
    for b in range(B):
        x1 = jax.nn.sigmoid(e_ref[b]) * xn_ref[b]
        h = _dot(x1, w0_ref[...], ((1,), (1,)))
        hs.append(h)
        ssum = ssum + jnp.sum(h, axis=0, keepdims=True)
        qsum = qsum + jnp.sum(h * h, axis=0, keepdims=True)
    mean = ssum / (B * N)
    var = qsum / (B * N) - mean * mean
    scale = g0_ref[...] / jnp.sqrt(var + 1e-5)
    shift = b0_ref[...] - mean * scale

    sig = jax.nn.sigmoid(sigp_ref[...])
    inv2 = 1.0 / (sig * sig)
    anc = anc_ref[...]
    a1 = anc * inv2
    ones_c = jnp.ones((C, 1), jnp.float32)
    c0 = _dot(anc * a1, ones_c, ((1,), (0,)))
    ones_col = jnp.ones((N, 1), jnp.float32)

    for b in range(B):
        hn = jnp.maximum(hs[b] * scale + shift, 0.0)
        t1 = _dot(inv2, hn * hn, ((1,), (1,)))
        t2 = _dot(a1, hn, ((1,), (1,)))
        logits = -0.5 * t1 + t2 - 0.5 * c0
        m = jnp.max(logits, axis=0, keepdims=True)
        e = jnp.exp(logits - m)
        sa = e / jnp.sum(e, axis=0, keepdims=True)
        den = _dot(sa, ones_col, ((1,), (0,)))
        sxh = _dot(sa, hn, ((1,), (0,)))
        nodes = (sxh - anc * den) / sig / (den + 1e-9)
        rn = jnp.sqrt(jnp.sum(nodes * nodes, axis=1, keepdims=True))
        nodes = nodes / jnp.maximum(rn, 1e-12)
        fl = jnp.sqrt(jnp.sum(nodes * nodes, keepdims=True))
        nodes = nodes / jnp.maximum(fl, 1e-12)
        nodes_ref[b] = nodes


def _d2_body(xc_ref, m1_ref, d2_ref):
    B, C, N = xc_ref.shape

    ones_c = jnp.ones((1, C), jnp.float32)
    ones_c_col = jnp.ones((C, 1), jnp.float32)
    for b in range(B):
        xc = xc_ref[b]
        m1 = m1_ref[b]
        mv = _dot(m1, xc, ((0,), (0,)))
        xsq = _dot(ones_c, xc * xc, ((1,), (0,)))
        vsq = _dot(m1 * m1, ones_c_col, ((0,), (0,)))
        d2_ref[b] = xsq - 2.0 * mv + vsq


def _tree_min(vals):
    while len(vals) > 1:
        vals = [jnp.minimum(vals[i], vals[i + 1])
                for i in range(0, len(vals) - 1, 2)] \
            + ([vals[-1]] if len(vals) % 2 else [])
    return vals[0]


def _sc_rank(d2):
    B, K, N = d2.shape
    info = plsc.get_sparse_core_info()
    nw = info.num_cores * info.num_subcores
    chunk = (B * N) // nw
    nchunk = N // chunk
    ngrp = chunk // info.num_lanes

    mesh = plsc.VectorSubcoreMesh(core_axis_name="c", subcore_axis_name="s")

    @functools.partial(
        pl.kernel, mesh=mesh,
        out_type=jax.ShapeDtypeStruct((B, _KNN, N), jnp.float32),
        scratch_types=[
            pltpu.VMEM((K, chunk), jnp.float32),
            pltpu.VMEM((_KNN, chunk), jnp.float32),
        ],
    )
    def rank_kernel(d2_hbm, li_hbm, dbuf, libuf):
        wid = jax.lax.axis_index("s") * info.num_cores + jax.lax.axis_index("c")
        b = wid // nchunk
        base = (wid % nchunk) * chunk
        pltpu.sync_copy(d2_hbm.at[b, :, pl.ds(base, chunk)], dbuf)

        def group(g, carry):
            off = g * info.num_lanes
            dwork = [dbuf[k, pl.ds(off, info.num_lanes)] for k in range(K)]
            for r in range(_KNN):
                mn = _tree_min(dwork)
                li = jnp.full((info.num_lanes,), float(K), jnp.float32)
                for k in reversed(range(K)):
                    li = jnp.where(dwork[k] == mn, float(k), li)
                for k in range(K):
                    dwork[k] = jnp.where(li == float(k), jnp.inf, dwork[k])
                libuf[r, pl.ds(off, info.num_lanes)] = li
            return carry

        jax.lax.fori_loop(0, ngrp, group, 0)
        pltpu.sync_copy(libuf, li_hbm.at[b, :, pl.ds(base, chunk)])

    return rank_kernel(d2)


def _edgeconv_body(xn_ref, m1_ref, w1a_ref, wd_ref, ids_ref, g1_ref, b1_ref,
                   out_ref):
    B, N, C = xn_ref.shape
    K = m1_ref.shape[2]

    lane = jax.lax.broadcasted_iota(jnp.int32, (N, K), 1).astype(jnp.float32)
    lane5 = jax.lax.broadcasted_iota(jnp.int32, (N, _KNN), 1).astype(jnp.float32)

    qs, sums, mxs, mns = [], [], [], []
    s1 = jnp.zeros((1, C), jnp.float32)
    s2 = jnp.zeros((1, C), jnp.float32)
    for b in range(B):
        q = _dot(xn_ref[b], wd_ref[...], ((1,), (1,)))
        pm = _dot(w1a_ref[...], m1_ref[b], ((1,), (0,)))
        qs.append(q)
        ids = ids_ref[b]

        ssum = jnp.zeros((N, C), jnp.float32)
        smax = jnp.full((N, C), -jnp.inf, jnp.float32)
        smin = jnp.full((N, C), jnp.inf, jnp.float32)
        cnt = jnp.zeros((1, K), jnp.float32)
        for m in range(_KNN):
            col = jnp.sum(jnp.where(lane5 == float(m), ids, 0.0), axis=1,
                          keepdims=True)
            mf = (lane == col).astype(jnp.float32)
            g = _dot(mf, pm, ((1,), (1,)))
            ssum = ssum + g
            smax = jnp.maximum(smax, g)
            smin = jnp.minimum(smin, g)
            cnt = cnt + jnp.sum(mf, axis=0, keepdims=True)
        sums.append(ssum)
        mxs.append(smax)
        mns.append(smin)
        s1 = s1 + jnp.sum(ssum, axis=0, keepdims=True) \
            + _KNN * jnp.sum(q, axis=0, keepdims=True)
        s2 = s2 + _dot(cnt, pm * pm, ((1,), (1,))) \
            + 2.0 * jnp.sum(q * ssum, axis=0, keepdims=True) \
            + _KNN * jnp.sum(q * q, axis=0, keepdims=True)

    count = B * N * _KNN
    mean = s1 / count
    var = s2 / count - mean * mean
    a = g1_ref[...] / jnp.sqrt(var + 1e-5)
    bb = b1_ref[...] - mean * a
    for b in range(B):
        meff = jnp.where(a >= 0.0, mxs[b], mns[b])
        y = jnp.maximum(a * (meff + qs[b]) + bb, 0.0)
        out_ref[b] = xn_ref[b] + y


def _run(interpret=False):
    def go(xn, xc, en, w0, g0, b0, anc, sigp, w1a, wd, g1, b1):
        B, N, C = xn.shape
        K = anc.shape[0]
        nodes = pl.pallas_call(
            _stage1_body,
            out_shape=jax.ShapeDtypeStruct((B, K, C), jnp.float32),
            interpret=interpret,
        )(xn, en, w0, g0, b0, anc, sigp)
        m1 = nodes.reshape(B, C, K)

        d2 = pl.pallas_call(
            _d2_body,
            out_shape=jax.ShapeDtypeStruct((B, K, N), jnp.float32),
            interpret=interpret,
        )(xc, m1)
        li = _sc_rank(d2)
        ids = li.reshape(B, N, _KNN)

        outn = pl.pallas_call(
            _edgeconv_body,
            out_shape=jax.ShapeDtypeStruct((B, N, C), jnp.float32),
            interpret=interpret,
        )(xn, m1, w1a, wd, ids, g1, b1)
        return outn
    return go


def kernel(x, edge, W0, gamma0, beta0, anchor, sigma_p, W1, gamma1, beta1):
    B, C, H, W = x.shape
    N = H * W
    xc = x.reshape(B, C, N)
    xn = xc.transpose(0, 2, 1)
    en = edge.reshape(B, N, 1)
    w1a = W1[:, :C]
    wd = W1[:, C:] - w1a
    outn = _run()(xn, xc, en, W0, gamma0.reshape(1, C), beta0.reshape(1, C),
                  anchor, sigma_p, w1a, wd,
                  gamma1.reshape(1, C), beta1.reshape(1, C))
    return outn.transpose(0, 2, 1).reshape(B, C, H, W)


# speedup vs baseline: 1.0057x; 1.0057x over previous
import functools

import jax
import jax.numpy as jnp
from jax.experimental import pallas as pl
from jax.experimental.pallas import tpu as pltpu
from jax.experimental.pallas import tpu_sc as plsc

_KNN = 5
_HIGH = jax.lax.Precision.HIGHEST


def _dot(a, b, dims):
    return jax.lax.dot_general(
        a, b, (dims, ((), ())),
        preferred_element_type=jnp.float32, precision=_HIGH)


def _stage1_body(xn_ref, e_ref, w0_ref, g0_ref, b0_ref, anc_ref, sigp_ref,
                 nodes_ref):
    B, N, C = xn_ref.shape

    hs = []
    ssum = jnp.zeros((1, C), jnp.float32)
    qsum = jnp.zeros((1, C), jnp.float32)
    for b in range(B):
        x1 = jax.nn.sigmoid(e_ref[b]) * xn_ref[b]
        h = _dot(x1, w0_ref[...], ((1,), (1,)))
        hs.append(h)
        ssum = ssum + jnp.sum(h, axis=0, keepdims=True)
        qsum = qsum + jnp.sum(h * h, axis=0, keepdims=True)
    mean = ssum / (B * N)
    var = qsum / (B * N) - mean * mean
    scale = g0_ref[...] / jnp.sqrt(var + 1e-5)
    shift = b0_ref[...] - mean * scale

    sig = jax.nn.sigmoid(sigp_ref[...])
    inv2 = 1.0 / (sig * sig)
    anc = anc_ref[...]
    a1 = anc * inv2
    ones_c = jnp.ones((C, 1), jnp.float32)
    c0 = _dot(anc * a1, ones_c, ((1,), (0,)))
    ones_col = jnp.ones((N, 1), jnp.float32)

    for b in range(B):
        hn = jnp.maximum(hs[b] * scale + shift, 0.0)
        t1 = _dot(inv2, hn * hn, ((1,), (1,)))
        t2 = _dot(a1, hn, ((1,), (1,)))
        logits = -0.5 * t1 + t2 - 0.5 * c0
        m = jnp.max(logits, axis=0, keepdims=True)
        e = jnp.exp(logits - m)
        sa = e / jnp.sum(e, axis=0, keepdims=True)
        den = _dot(sa, ones_col, ((1,), (0,)))
        sxh = _dot(sa, hn, ((1,), (0,)))
        nodes = (sxh - anc * den) / sig / (den + 1e-9)
        rn = jnp.sqrt(jnp.sum(nodes * nodes, axis=1, keepdims=True))
        nodes = nodes / jnp.maximum(rn, 1e-12)
        fl = jnp.sqrt(jnp.sum(nodes * nodes, keepdims=True))
        nodes = nodes / jnp.maximum(fl, 1e-12)
        nodes_ref[b] = nodes


def _d2_body(xc_ref, m1_ref, d2_ref):
    B, C, N = xc_ref.shape

    ones_c = jnp.ones((1, C), jnp.float32)
    ones_c_col = jnp.ones((C, 1), jnp.float32)
    for b in range(B):
        xc = xc_ref[b]
        m1 = m1_ref[b]
        mv = _dot(m1, xc, ((0,), (0,)))
        xsq = _dot(ones_c, xc * xc, ((1,), (0,)))
        vsq = _dot(m1 * m1, ones_c_col, ((0,), (0,)))
        d2_ref[b] = xsq - 2.0 * mv + vsq


def _tree_min(vals):
    while len(vals) > 1:
        vals = [jnp.minimum(vals[i], vals[i + 1])
                for i in range(0, len(vals) - 1, 2)] \
            + ([vals[-1]] if len(vals) % 2 else [])
    return vals[0]


def _sc_rank(d2):
    B, K, N = d2.shape
    info = plsc.get_sparse_core_info()
    nw = info.num_cores * info.num_subcores
    chunk = (B * N) // nw
    nchunk = N // chunk
    ngrp = chunk // info.num_lanes

    mesh = plsc.VectorSubcoreMesh(core_axis_name="c", subcore_axis_name="s")

    @functools.partial(
        pl.kernel, mesh=mesh,
        out_type=jax.ShapeDtypeStruct((B, _KNN, N), jnp.float32),
        scratch_types=[
            pltpu.VMEM((K, chunk), jnp.float32),
            pltpu.VMEM((_KNN, chunk), jnp.float32),
        ],
    )
    def rank_kernel(d2_hbm, li_hbm, dbuf, libuf):
        wid = jax.lax.axis_index("s") * info.num_cores + jax.lax.axis_index("c")
        b = wid // nchunk
        base = (wid % nchunk) * chunk
        pltpu.sync_copy(d2_hbm.at[b, :, pl.ds(base, chunk)], dbuf)

        def group(g, carry):
            off = g * info.num_lanes
            dwork = [dbuf[k, pl.ds(off, info.num_lanes)] for k in range(K)]
            for r in range(_KNN):
                mn = _tree_min(dwork)
                li = jnp.full((info.num_lanes,), float(K), jnp.float32)
                for k in reversed(range(K)):
                    li = jnp.where(dwork[k] == mn, float(k), li)
                for k in range(K):
                    dwork[k] = jnp.where(li == float(k), jnp.inf, dwork[k])
                libuf[r, pl.ds(off, info.num_lanes)] = li
            return carry

        jax.lax.fori_loop(0, ngrp, group, 0)
        pltpu.sync_copy(libuf, li_hbm.at[b, :, pl.ds(base, chunk)])

    return rank_kernel(d2)


def _edgeconv_body(xn_ref, m1_ref, w1a_ref, wd_ref, ids_ref, g1_ref, b1_ref,
                   out_ref):
    B, N, C = xn_ref.shape
    K = m1_ref.shape[2]

    kio = jax.lax.broadcasted_iota(jnp.int32, (K, N), 0).astype(jnp.float32)
    ones_n = jnp.ones((1, N), jnp.float32)

    qs, sums, mxs, mns = [], [], [], []
    s1 = jnp.zeros((1, C), jnp.float32)
    s2 = jnp.zeros((1, C), jnp.float32)
    for b in range(B):
        q = _dot(xn_ref[b], wd_ref[...], ((1,), (1,)))
        pmt = _dot(m1_ref[b], w1a_ref[...], ((0,), (1,)))
        qs.append(q)
        idsb = ids_ref[b]

        ssum = jnp.zeros((N, C), jnp.float32)
        smax = jnp.full((N, C), -jnp.inf, jnp.float32)
        smin = jnp.full((N, C), jnp.inf, jnp.float32)
        mfall = jnp.zeros((K, N), jnp.float32)
        for m in range(_KNN):
            mft = (kio == idsb[m:m + 1, :]).astype(jnp.float32)
            g = _dot(mft, pmt, ((0,), (0,)))
            ssum = ssum + g
            smax = jnp.maximum(smax, g)
            smin = jnp.minimum(smin, g)
            mfall = mfall + mft
        sums.append(ssum)
        mxs.append(smax)
        mns.append(smin)
        cnt = _dot(ones_n, mfall, ((1,), (1,)))
        s1 = s1 + jnp.sum(ssum, axis=0, keepdims=True) \
            + _KNN * jnp.sum(q, axis=0, keepdims=True)
        s2 = s2 + _dot(cnt, pmt * pmt, ((1,), (0,))) \
            + 2.0 * jnp.sum(q * ssum, axis=0, keepdims=True) \
            + _KNN * jnp.sum(q * q, axis=0, keepdims=True)

    count = B * N * _KNN
    mean = s1 / count
    var = s2 / count - mean * mean
    a = g1_ref[...] / jnp.sqrt(var + 1e-5)
    bb = b1_ref[...] - mean * a
    for b in range(B):
        meff = jnp.where(a >= 0.0, mxs[b], mns[b])
        y = jnp.maximum(a * (meff + qs[b]) + bb, 0.0)
        out_ref[b] = xn_ref[b] + y


def _run(interpret=False):
    def go(xn, xc, en, w0, g0, b0, anc, sigp, w1a, wd, g1, b1):
        B, N, C = xn.shape
        K = anc.shape[0]
        nodes = pl.pallas_call(
            _stage1_body,
            out_shape=jax.ShapeDtypeStruct((B, K, C), jnp.float32),
            interpret=interpret,
        )(xn, en, w0, g0, b0, anc, sigp)
        m1 = nodes.reshape(B, C, K)

        d2 = pl.pallas_call(
            _d2_body,
            out_shape=jax.ShapeDtypeStruct((B, K, N), jnp.float32),
            interpret=interpret,
        )(xc, m1)
        li = _sc_rank(d2)
        ids = li.reshape(B, N, _KNN).transpose(0, 2, 1)

        outn = pl.pallas_call(
            _edgeconv_body,
            out_shape=jax.ShapeDtypeStruct((B, N, C), jnp.float32),
            interpret=interpret,
        )(xn, m1, w1a, wd, ids, g1, b1)
        return outn
    return go


def kernel(x, edge, W0, gamma0, beta0, anchor, sigma_p, W1, gamma1, beta1):
    B, C, H, W = x.shape
    N = H * W
    xc = x.reshape(B, C, N)
    xn = xc.transpose(0, 2, 1)
    en = edge.reshape(B, N, 1)
    w1a = W1[:, :C]
    wd = W1[:, C:] - w1a
    outn = _run()(xn, xc, en, W0, gamma0.reshape(1, C), beta0.reshape(1, C),
                  anchor, sigma_p, w1a, wd,
                  gamma1.reshape(1, C), beta1.reshape(1, C))
    return outn.transpose(0, 2, 1).reshape(B, C, H, W)
